# baseline (speedup 1.0000x reference)
import functools

import jax
import jax.numpy as jnp
from jax import lax
from jax.experimental import pallas as pl
from jax.experimental.pallas import tpu as pltpu
from jax.experimental.pallas import tpu_sc as plsc


def _matmul_table(h_paper, h_conf_pad):
    m, d = h_paper.shape
    c, _ = h_conf_pad.shape
    npair = c // 256

    def body(a_ref, b_ref, o_ref):
        a = a_ref[...]
        r1 = lax.dot_general(a, b_ref[:128], (((1,), (1,)), ((), ())),
                             preferred_element_type=jnp.float32)
        r2 = lax.dot_general(a, b_ref[128:], (((1,), (1,)), ((), ())),
                             preferred_element_type=jnp.float32)
        b1 = lax.bitcast_convert_type(r1, jnp.uint32)
        b2 = lax.bitcast_convert_type(r2, jnp.uint32)
        lo = (b1 + jnp.uint32(0x8000)) >> 16
        hi = (b2 + jnp.uint32(0x8000)) & jnp.uint32(0xFFFF0000)
        o_ref[...] = lax.bitcast_convert_type(lo | hi, jnp.int32)[None]

    return pl.pallas_call(
        body,
        grid=(npair,),
        in_specs=[
            pl.BlockSpec((m, d), lambda j: (0, 0)),
            pl.BlockSpec((256, d), lambda j: (j, 0)),
        ],
        out_specs=pl.BlockSpec((1, m, 128), lambda j: (j, 0, 0)),
        out_shape=jax.ShapeDtypeStruct((npair, m, 128), jnp.int32),
    )(h_paper, h_conf_pad)


_NW = 32
_CH = 128


def _ragged(e):
    nch_total = e // _CH
    nch_lo = nch_total // _NW
    extra = nch_total - nch_lo * _NW
    return nch_lo, extra, (nch_lo + 1) * _CH


def _compute_indices(src_idx, dst_idx, n_rows):
    e = src_idx.shape[0]
    plane = n_rows * 128
    nch_lo, extra, cap = _ragged(e)
    mesh = plsc.VectorSubcoreMesh(core_axis_name="c", subcore_axis_name="s")

    @functools.partial(
        pl.kernel,
        mesh=mesh,
        out_type=jax.ShapeDtypeStruct((1, e), jnp.int32),
        scratch_types=[
            pltpu.VMEM((cap,), jnp.int32),
            pltpu.VMEM((cap,), jnp.int32),
            pltpu.VMEM((cap,), jnp.int32),
        ],
    )
    def k(src_hbm, dst_hbm, out_hbm, src_v, dst_v, idx_v):
        wid = lax.axis_index("s") * 2 + lax.axis_index("c")
        nch = jnp.where(wid < extra, nch_lo + 1, nch_lo)
        base = (wid * nch_lo + jnp.minimum(wid, extra)) * _CH
        pltpu.sync_copy(src_hbm.at[pl.ds(base, nch_lo * _CH)],
                        src_v.at[pl.ds(0, nch_lo * _CH)])
        pltpu.sync_copy(dst_hbm.at[pl.ds(base, nch_lo * _CH)],
                        dst_v.at[pl.ds(0, nch_lo * _CH)])

        @pl.when(wid < extra)
        def _():
            pltpu.sync_copy(src_hbm.at[pl.ds(base + nch_lo * _CH, _CH)],
                            src_v.at[pl.ds(nch_lo * _CH, _CH)])
            pltpu.sync_copy(dst_hbm.at[pl.ds(base + nch_lo * _CH, _CH)],
                            dst_v.at[pl.ds(nch_lo * _CH, _CH)])

        def chunk(j, carry):
            for kk in range(_CH // 16):
                off = j * _CH + kk * 16
                s = src_v[pl.ds(off, 16)]
                t = dst_v[pl.ds(off, 16)]
                idx_v[pl.ds(off, 16)] = (t >> 8) * plane + s * 128 + (t & 127)
            return carry

        lax.fori_loop(0, nch, chunk, 0)
        pltpu.sync_copy(idx_v.at[pl.ds(0, nch_lo * _CH)],
                        out_hbm.at[0, pl.ds(base, nch_lo * _CH)])

        @pl.when(wid < extra)
        def _():
            pltpu.sync_copy(idx_v.at[pl.ds(nch_lo * _CH, _CH)],
                            out_hbm.at[0, pl.ds(base + nch_lo * _CH, _CH)])

    return k(src_idx, dst_idx)


def _gather_scores(q_flat, idx2d, dst_idx):
    e = dst_idx.shape[0]
    nch_lo, extra, cap = _ragged(e)
    mesh = plsc.VectorSubcoreMesh(core_axis_name="c", subcore_axis_name="s")

    @functools.partial(
        pl.kernel,
        mesh=mesh,
        out_type=jax.ShapeDtypeStruct((1, e), jnp.float32),
        scratch_types=[
            pltpu.VMEM((cap,), jnp.int32),
            pltpu.VMEM((cap,), jnp.int32),
            pltpu.VMEM((cap,), jnp.int32),
            pltpu.VMEM((cap,), jnp.float32),
            pltpu.SemaphoreType.DMA,
        ],
    )
    def k(q_hbm, idx_hbm, dst_hbm, out_hbm, idx_v, dst_v, w_v, f_v, sem):
        wid = lax.axis_index("s") * 2 + lax.axis_index("c")
        nch = jnp.where(wid < extra, nch_lo + 1, nch_lo)
        base = (wid * nch_lo + jnp.minimum(wid, extra)) * _CH
        pltpu.sync_copy(idx_hbm.at[0, pl.ds(base, nch_lo * _CH)],
                        idx_v.at[pl.ds(0, nch_lo * _CH)])

        @pl.when(wid < extra)
        def _():
            pltpu.sync_copy(idx_hbm.at[0, pl.ds(base + nch_lo * _CH, _CH)],
                            idx_v.at[pl.ds(nch_lo * _CH, _CH)])

        def fire(j, carry):
            pltpu.async_copy(
                q_hbm.at[idx_v.at[pl.ds(j * _CH, _CH)]],
                w_v.at[pl.ds(j * _CH, _CH)], sem
            )
            return carry

        lax.fori_loop(0, nch, fire, 0)

        pltpu.sync_copy(dst_hbm.at[pl.ds(base, nch_lo * _CH)],
                        dst_v.at[pl.ds(0, nch_lo * _CH)])

        @pl.when(wid < extra)
        def _():
            pltpu.sync_copy(dst_hbm.at[pl.ds(base + nch_lo * _CH, _CH)],
                            dst_v.at[pl.ds(nch_lo * _CH, _CH)])

        pltpu.make_async_copy(
            dst_hbm.at[pl.ds(base, nch_lo * _CH)],
            w_v.at[pl.ds(0, nch_lo * _CH)], sem
        ).wait()

        @pl.when(wid < extra)
        def _():
            pltpu.make_async_copy(
                dst_hbm.at[pl.ds(base, _CH)], w_v.at[pl.ds(0, _CH)], sem
            ).wait()

        def expand(j, carry):
            for kk in range(_CH // 16):
                off = j * _CH + kk * 16
                w = w_v[pl.ds(off, 16)]
                t = dst_v[pl.ds(off, 16)]
                odd = (t & 128) != 0
                bits = jnp.where(odd, w & jnp.int32(-65536), w << 16)
                f_v[pl.ds(off, 16)] = lax.bitcast_convert_type(
                    bits, jnp.float32)
            return carry

        lax.fori_loop(0, nch, expand, 0)
        pltpu.sync_copy(f_v.at[pl.ds(0, nch_lo * _CH)],
                        out_hbm.at[0, pl.ds(base, nch_lo * _CH)])

        @pl.when(wid < extra)
        def _():
            pltpu.sync_copy(f_v.at[pl.ds(nch_lo * _CH, _CH)],
                            out_hbm.at[0, pl.ds(base + nch_lo * _CH, _CH)])

    return k(q_flat, idx2d, dst_idx)


def kernel(h_paper, h_conf, src_idx, dst_idx):
    n_conf, d = h_conf.shape
    c_pad = 1024
    h_conf_pad = jnp.pad(h_conf, ((0, c_pad - n_conf), (0, 0)))
    q = _matmul_table(h_paper, h_conf_pad)
    idx2d = _compute_indices(src_idx, dst_idx, h_paper.shape[0])
    out = _gather_scores(q.reshape(-1), idx2d, dst_idx)
    return out.reshape(-1, 1)


# speedup vs baseline: 1.0027x; 1.0027x over previous
import functools

import jax
import jax.numpy as jnp
from jax import lax
from jax.experimental import pallas as pl
from jax.experimental.pallas import tpu as pltpu
from jax.experimental.pallas import tpu_sc as plsc


def _matmul_table(h_paper, h_conf_pad):
    m, d = h_paper.shape
    c, _ = h_conf_pad.shape
    npair = c // 256

    def body(a_ref, b_ref, o_ref):
        a = a_ref[...]
        r1 = lax.dot_general(a, b_ref[:128], (((1,), (1,)), ((), ())),
                             preferred_element_type=jnp.float32)
        r2 = lax.dot_general(a, b_ref[128:], (((1,), (1,)), ((), ())),
                             preferred_element_type=jnp.float32)
        b1 = lax.bitcast_convert_type(r1, jnp.uint32)
        b2 = lax.bitcast_convert_type(r2, jnp.uint32)
        lo = (b1 + jnp.uint32(0x8000)) >> 16
        hi = (b2 + jnp.uint32(0x8000)) & jnp.uint32(0xFFFF0000)
        o_ref[...] = lax.bitcast_convert_type(lo | hi, jnp.int32)[None]

    return pl.pallas_call(
        body,
        grid=(npair,),
        in_specs=[
            pl.BlockSpec((m, d), lambda j: (0, 0)),
            pl.BlockSpec((256, d), lambda j: (j, 0)),
        ],
        out_specs=pl.BlockSpec((1, m, 128), lambda j: (j, 0, 0)),
        out_shape=jax.ShapeDtypeStruct((npair, m, 128), jnp.int32),
    )(h_paper, h_conf_pad)


def _gather_scores(q_flat, src_idx, dst_idx, n_rows):
    e = src_idx.shape[0]
    plane = n_rows * 128
    nw = 32
    ch = 128
    nch_total = e // ch
    nch_lo = nch_total // nw
    extra = nch_total - nch_lo * nw
    nch_hi = nch_lo + 1
    cap = nch_hi * ch
    mesh = plsc.VectorSubcoreMesh(core_axis_name="c", subcore_axis_name="s")

    @functools.partial(
        pl.kernel,
        mesh=mesh,
        out_type=jax.ShapeDtypeStruct((1, e), jnp.float32),
        scratch_types=[
            pltpu.VMEM((cap,), jnp.int32),
            pltpu.VMEM((cap,), jnp.int32),
            pltpu.VMEM((nch_hi, ch), jnp.int32),
            pltpu.VMEM((cap,), jnp.int32),
            pltpu.VMEM((cap,), jnp.float32),
            pltpu.SemaphoreType.DMA,
        ],
    )
    def k(q_hbm, src_hbm, dst_hbm, out_hbm,
          src_v, dst_v, idx_v, w_v, f_v, sem):
        wid = lax.axis_index("s") * 2 + lax.axis_index("c")
        nch = jnp.where(wid < extra, nch_hi, nch_lo)
        base = (wid * nch_lo + jnp.minimum(wid, extra)) * ch
        pltpu.sync_copy(src_hbm.at[pl.ds(base, nch_lo * ch)],
                        src_v.at[pl.ds(0, nch_lo * ch)])
        pltpu.sync_copy(dst_hbm.at[pl.ds(base, nch_lo * ch)],
                        dst_v.at[pl.ds(0, nch_lo * ch)])

        @pl.when(wid < extra)
        def _():
            pltpu.sync_copy(src_hbm.at[pl.ds(base + nch_lo * ch, ch)],
                            src_v.at[pl.ds(nch_lo * ch, ch)])
            pltpu.sync_copy(dst_hbm.at[pl.ds(base + nch_lo * ch, ch)],
                            dst_v.at[pl.ds(nch_lo * ch, ch)])

        def chunk(j, carry):
            for kk in range(ch // 16):
                off = j * ch + kk * 16
                s = src_v[pl.ds(off, 16)]
                t = dst_v[pl.ds(off, 16)]
                idx_v[j, pl.ds(kk * 16, 16)] = (
                    (t >> 8) * plane + s * 128 + (t & 127)
                )
            pltpu.async_copy(
                q_hbm.at[idx_v.at[j]], w_v.at[pl.ds(j * ch, ch)], sem
            )
            return carry

        lax.fori_loop(0, nch, chunk, 0)

        pltpu.make_async_copy(
            src_hbm.at[pl.ds(base, nch_lo * ch)],
            w_v.at[pl.ds(0, nch_lo * ch)], sem
        ).wait()

        @pl.when(wid < extra)
        def _():
            pltpu.make_async_copy(
                src_hbm.at[pl.ds(base, ch)], w_v.at[pl.ds(0, ch)], sem
            ).wait()

        def expand(j, carry):
            for kk in range(ch // 16):
                off = j * ch + kk * 16
                w = w_v[pl.ds(off, 16)]
                t = dst_v[pl.ds(off, 16)]
                odd = (t & 128) != 0
                bits = jnp.where(odd, w & jnp.int32(-65536), w << 16)
                f_v[pl.ds(off, 16)] = lax.bitcast_convert_type(
                    bits, jnp.float32)
            return carry

        lax.fori_loop(0, nch, expand, 0)
        pltpu.sync_copy(f_v.at[pl.ds(0, nch_lo * ch)],
                        out_hbm.at[0, pl.ds(base, nch_lo * ch)])

        @pl.when(wid < extra)
        def _():
            pltpu.sync_copy(f_v.at[pl.ds(nch_lo * ch, ch)],
                            out_hbm.at[0, pl.ds(base + nch_lo * ch, ch)])

    return k(q_flat, src_idx, dst_idx)


def kernel(h_paper, h_conf, src_idx, dst_idx):
    n_conf, d = h_conf.shape
    c_pad = 1024
    h_conf_pad = jnp.pad(h_conf, ((0, c_pad - n_conf), (0, 0)))
    q = _matmul_table(h_paper, h_conf_pad)
    out = _gather_scores(q.reshape(-1), src_idx, dst_idx, h_paper.shape[0])
    return out.reshape(-1, 1)
